# Initial kernel scaffold; baseline (speedup 1.0000x reference)
import jax
import jax.numpy as jnp
from jax.experimental import pallas as pl


def kernel(x, edge_index, W1, as1, ad1, b1, W2, as2, ad2, b2, W3, as3, ad3, b3):
    raise NotImplementedError("write your pallas kernel here")


# speedup vs baseline: 10.8789x; 10.8789x over previous
import functools
import jax
import jax.numpy as jnp
from jax import lax
from jax.experimental import pallas as pl
from jax.experimental.pallas import tpu as pltpu
from jax.experimental.pallas import tpu_sc as plsc

_N = 10000
_E = 320000
_IN = 128
_OUT = 64
_HEADS = 4
_NP = 10240
_EP = 327680
_LANE = 16

_f32 = jnp.float32
_i32 = jnp.int32


def _iota16():
    return lax.iota(_i32, _LANE)


def _full16(v):
    return jnp.full((_LANE,), v, _i32)


def _proj1_body(x_ref, w_ref, as_ref, ad_ref, tab_ref, ab_ref):
    h = jnp.dot(x_ref[...], w_ref[...], preferred_element_type=_f32)
    for hd in range(_HEADS):
        hh = h[:, hd * 128:(hd + 1) * 128]
        tab_ref[hd] = hh
        ab_ref[:, hd:hd + 1] = jnp.sum(hh * as_ref[hd][None, :], axis=1,
                                       keepdims=True)
        ab_ref[:, _HEADS + hd:_HEADS + hd + 1] = jnp.sum(
            hh * ad_ref[hd][None, :], axis=1, keepdims=True)


def _proj1(xp, W1, as1, ad1):
    R = 2048
    grid = (_NP // R,)
    return pl.pallas_call(
        _proj1_body,
        grid=grid,
        in_specs=[
            pl.BlockSpec((R, _IN), lambda i: (i, 0)),
            pl.BlockSpec((_IN, _HEADS * 128), lambda i: (0, 0)),
            pl.BlockSpec((_HEADS, 128), lambda i: (0, 0)),
            pl.BlockSpec((_HEADS, 128), lambda i: (0, 0)),
        ],
        out_specs=[
            pl.BlockSpec((_HEADS, R, 128), lambda i: (0, i, 0)),
            pl.BlockSpec((R, 2 * _HEADS), lambda i: (i, 0)),
        ],
        out_shape=[
            jax.ShapeDtypeStruct((_HEADS, _NP, 128), _f32),
            jax.ShapeDtypeStruct((_NP, 2 * _HEADS), _f32),
        ],
    )(xp, W1, as1, ad1)


def _proj23_body(num_ref, b1_ref, w2_ref, w3_ref, as_ref, ad_ref,
                 tab_ref, ab_ref):
    hs = []
    for hd in range(_HEADS):
        blk = num_ref[hd]
        v = blk[:, :128] / (blk[:, 128:129] + 1e-16)
        v = v + b1_ref[0, hd * 128:(hd + 1) * 128][None, :]
        hs.append(v)
    h1 = jnp.concatenate(hs, axis=1)
    h1 = jnp.where(h1 > 0, h1, jnp.exp(jnp.minimum(h1, 0.0)) - 1.0)
    t2 = jnp.dot(h1, w2_ref[...], preferred_element_type=_f32)
    t3 = jnp.dot(h1, w3_ref[...], preferred_element_type=_f32)
    tab_ref[0] = t2
    tab_ref[1] = t3
    ab_ref[:, 0:1] = jnp.sum(t2 * as_ref[0][None, :], 1, keepdims=True)
    ab_ref[:, 1:2] = jnp.sum(t3 * as_ref[1][None, :], 1, keepdims=True)
    ab_ref[:, 2:3] = jnp.sum(t2 * ad_ref[0][None, :], 1, keepdims=True)
    ab_ref[:, 3:4] = jnp.sum(t3 * ad_ref[1][None, :], 1, keepdims=True)


def _proj23(num1, b1, W2, W3, as2, ad2, as3, ad3):
    R = 2048
    CP = num1.shape[-1]
    D = _HEADS * 2 * _OUT
    as23 = jnp.concatenate([as2, as3], axis=0)
    ad23 = jnp.concatenate([ad2, ad3], axis=0)
    return pl.pallas_call(
        _proj23_body,
        grid=(_NP // R,),
        in_specs=[
            pl.BlockSpec((_HEADS, R, CP), lambda i: (0, i, 0)),
            pl.BlockSpec((1, D), lambda i: (0, 0)),
            pl.BlockSpec((D, _OUT), lambda i: (0, 0)),
            pl.BlockSpec((D, _OUT), lambda i: (0, 0)),
            pl.BlockSpec((2, _OUT), lambda i: (0, 0)),
            pl.BlockSpec((2, _OUT), lambda i: (0, 0)),
        ],
        out_specs=[
            pl.BlockSpec((2, R, _OUT), lambda i: (0, i, 0)),
            pl.BlockSpec((R, 4), lambda i: (i, 0)),
        ],
        out_shape=[
            jax.ShapeDtypeStruct((2, _NP, _OUT), _f32),
            jax.ShapeDtypeStruct((_NP, 4), _f32),
        ],
    )(num1, b1.reshape(1, D), W2, W3, as23, ad23)


def _fin23_body(num_ref, b_ref, out_ref):
    for hd in range(2):
        blk = num_ref[hd]
        out_ref[hd] = blk[:, :_OUT] / (blk[:, _OUT:_OUT + 1] + 1e-16) \
            + b_ref[hd][None, :]


def _fin23(num23, b23):
    R = 2048
    CP = num23.shape[-1]
    return pl.pallas_call(
        _fin23_body,
        grid=(_NP // R,),
        in_specs=[
            pl.BlockSpec((2, R, CP), lambda i: (0, i, 0)),
            pl.BlockSpec((2, _OUT), lambda i: (0, 0)),
        ],
        out_specs=pl.BlockSpec((2, R, _OUT), lambda i: (0, i, 0)),
        out_shape=jax.ShapeDtypeStruct((2, _NP, _OUT), _f32),
    )(num23, b23)


def _wts_body(nh, ba, per_tile, abt_ref, src_ref, dst_ref, w_ref,
              sidx, didx, abuf, wbuf, sem):
    core = lax.axis_index("c")
    sub = lax.axis_index("s")
    wid = sub * 2 + core
    nchunks = per_tile // ba

    def chunk(i, _):
        base = wid * per_tile + i * ba
        pltpu.sync_copy(src_ref.at[pl.ds(base, ba)], sidx)
        pltpu.sync_copy(dst_ref.at[pl.ds(base, ba)], didx)
        cps = []
        for h in range(nh):
            cps.append(pltpu.async_copy(
                abt_ref.at[h].at[sidx],
                abuf.at[pl.ds(h * ba, ba)], sem))
            cps.append(pltpu.async_copy(
                abt_ref.at[nh + h].at[didx],
                abuf.at[pl.ds((nh + h) * ba, ba)], sem))
        for cp in cps:
            cp.wait()

        def grp(g, _):
            rows = g * _LANE + _iota16()
            for h in range(nh):
                sa = plsc.load_gather(abuf, [h * ba + rows])
                da = plsc.load_gather(abuf, [(nh + h) * ba + rows])
                al = sa + da
                al = jnp.maximum(al, 0.2 * al)
                plsc.store_scatter(wbuf, [h * ba + rows], jnp.exp(al))
            return 0

        lax.fori_loop(0, ba // _LANE, grp, 0)
        for h in range(nh):
            pltpu.sync_copy(wbuf.at[pl.ds(h * ba, ba)],
                            w_ref.at[h].at[pl.ds(base, ba)])
        return 0

    lax.fori_loop(0, nchunks, chunk, 0)


def _wts(abt, src, dst, nh):
    e = src.shape[0]
    per_tile = e // 32
    ba = min(128, per_tile)
    assert per_tile % ba == 0 and ba % _LANE == 0 and ba % 128 == 0
    mesh = plsc.VectorSubcoreMesh(core_axis_name="c", subcore_axis_name="s")
    body = functools.partial(_wts_body, nh, ba, per_tile)
    return pl.kernel(
        body,
        out_type=jax.ShapeDtypeStruct((nh, e), _f32),
        mesh=mesh,
        compiler_params=pltpu.CompilerParams(use_tc_tiling_on_sc=False, needs_layout_passes=False),
        scratch_types=[
            pltpu.VMEM((ba,), _i32),
            pltpu.VMEM((ba,), _i32),
            pltpu.VMEM((2 * nh * ba,), _f32),
            pltpu.VMEM((nh * ba,), _f32),
            pltpu.SemaphoreType.DMA,
        ],
    )(abt, src, dst)


def _agg_body(nh, c, cp, b, per_tile, rows_tile,
              tab_ref, w_ref, src_ref, dst_ref, z_ref, num_ref,
              acc, gbuf, rbuf, sidx, didx, wchunk, sem):
    core = lax.axis_index("c")
    sub = lax.axis_index("s")
    hpc = nh // 2
    nchunks = per_tile // b

    for k in range(hpc):
        h = core * hpc + k
        pltpu.sync_copy(z_ref.at[pl.ds(sub * rows_tile, rows_tile)],
                        acc.at[pl.ds(sub * rows_tile, rows_tile)])
        plsc.subcore_barrier()

        def chunk(i, _):
            base = sub * per_tile + i * b
            pltpu.sync_copy(src_ref.at[pl.ds(base, b)], sidx)
            pltpu.sync_copy(dst_ref.at[pl.ds(base, b)], didx)
            pltpu.sync_copy(w_ref.at[h].at[pl.ds(base, b)], wchunk)
            pltpu.async_copy(tab_ref.at[h].at[sidx], gbuf, sem).wait()

            lane0 = (_iota16() == 0).astype(_f32)

            def edge(e2, _):
                wv = plsc.load_gather(wchunk, [_full16(e2)])
                for j in range(c // _LANE):
                    sl = pl.ds(j * _LANE, _LANE)
                    rbuf[e2, sl] = gbuf[e2, sl] * wv
                rbuf[e2, pl.ds(c, _LANE)] = wv * lane0
                return 0

            lax.fori_loop(0, b, edge, 0)
            pltpu.sync_copy(rbuf, acc.at[didx], add=True)
            return 0

        lax.fori_loop(0, nchunks, chunk, 0)
        plsc.subcore_barrier()
        pltpu.sync_copy(acc.at[pl.ds(sub * rows_tile, rows_tile)],
                        num_ref.at[h].at[pl.ds(sub * rows_tile, rows_tile)])
        plsc.subcore_barrier()


def _agg(tab, w, src, dst, c):
    nh = tab.shape[0]
    e = src.shape[0]
    cp = c + _LANE
    per_tile = e // 16
    b = min(128, per_tile)
    rows_tile = _NP // 16
    assert per_tile % b == 0 and b % _LANE == 0 and b % 128 == 0
    mesh = plsc.VectorSubcoreMesh(core_axis_name="c", subcore_axis_name="s")
    z = jnp.zeros((_NP, cp), _f32)
    body = functools.partial(_agg_body, nh, c, cp, b, per_tile, rows_tile)
    return pl.kernel(
        body,
        out_type=jax.ShapeDtypeStruct((nh, _NP, cp), _f32),
        mesh=mesh,
        compiler_params=pltpu.CompilerParams(use_tc_tiling_on_sc=False, needs_layout_passes=False),
        scratch_types=[
            pltpu.VMEM_SHARED((_NP, cp), _f32),
            pltpu.VMEM((b, c), _f32),
            pltpu.VMEM((b, cp), _f32),
            pltpu.VMEM((b,), _i32),
            pltpu.VMEM((b,), _i32),
            pltpu.VMEM((b,), _f32),
            pltpu.SemaphoreType.DMA,
        ],
    )(tab, w, src, dst, z)


def kernel(x, edge_index, W1, as1, ad1, b1, W2, as2, ad2, b2,
           W3, as3, ad3, b3):
    src = edge_index[0].astype(_i32)
    dst = edge_index[1].astype(_i32)
    pad = (jnp.arange(_EP - _E, dtype=_i32) % 192) + (_NP - 192)
    src = jnp.concatenate([src, pad])
    dst = jnp.concatenate([dst, pad])
    xp = jnp.zeros((_NP, _IN), _f32).at[:_N].set(x)

    tab1, ab1 = _proj1(xp, W1, as1, ad1)
    w1 = _wts(ab1.T, src, dst, _HEADS)
    num1 = _agg(tab1, w1, src, dst, 128)

    tab23, ab23 = _proj23(num1, b1, W2, W3, as2, ad2, as3, ad3)
    w23 = _wts(ab23.T, src, dst, 2)
    num23 = _agg(tab23, w23, src, dst, _OUT)

    out = _fin23(num23, jnp.stack([b2, b3]))
    return out[0, :_N], out[1, :_N]


# speedup vs baseline: 18.4439x; 1.6954x over previous
import functools
import jax
import jax.numpy as jnp
from jax import lax
from jax.experimental import pallas as pl
from jax.experimental.pallas import tpu as pltpu
from jax.experimental.pallas import tpu_sc as plsc

_N = 10000
_E = 320000
_IN = 128
_OUT = 64
_HEADS = 4
_NP = 10240
_EP = 327680
_LANE = 16

_f32 = jnp.float32
_i32 = jnp.int32


def _iota16():
    return lax.iota(_i32, _LANE)


def _full16(v):
    return jnp.full((_LANE,), v, _i32)


def _proj1_body(x_ref, w_ref, as_ref, ad_ref, tab_ref, ab_ref):
    h = jnp.dot(x_ref[...], w_ref[...], preferred_element_type=_f32)
    pad = jnp.zeros((h.shape[0], _LANE), _f32)
    for hd in range(_HEADS):
        hh = h[:, hd * 128:(hd + 1) * 128]
        tab_ref[hd] = jnp.concatenate([hh, pad], axis=1)
        ab_ref[:, hd:hd + 1] = jnp.sum(hh * as_ref[hd][None, :], axis=1,
                                       keepdims=True)
        ab_ref[:, _HEADS + hd:_HEADS + hd + 1] = jnp.sum(
            hh * ad_ref[hd][None, :], axis=1, keepdims=True)


def _proj1(xp, W1, as1, ad1):
    R = 2048
    grid = (_NP // R,)
    return pl.pallas_call(
        _proj1_body,
        grid=grid,
        in_specs=[
            pl.BlockSpec((R, _IN), lambda i: (i, 0)),
            pl.BlockSpec((_IN, _HEADS * 128), lambda i: (0, 0)),
            pl.BlockSpec((_HEADS, 128), lambda i: (0, 0)),
            pl.BlockSpec((_HEADS, 128), lambda i: (0, 0)),
        ],
        out_specs=[
            pl.BlockSpec((_HEADS, R, 128 + _LANE), lambda i: (0, i, 0)),
            pl.BlockSpec((R, 2 * _HEADS), lambda i: (i, 0)),
        ],
        out_shape=[
            jax.ShapeDtypeStruct((_HEADS, _NP, 128 + _LANE), _f32),
            jax.ShapeDtypeStruct((_NP, 2 * _HEADS), _f32),
        ],
    )(xp, W1, as1, ad1)


def _proj23_body(num_ref, b1_ref, w2_ref, w3_ref, as_ref, ad_ref,
                 tab_ref, ab_ref):
    hs = []
    for hd in range(_HEADS):
        blk = num_ref[hd]
        v = blk[:, :128] / (blk[:, 128:129] + 1e-16)
        v = v + b1_ref[0, hd * 128:(hd + 1) * 128][None, :]
        hs.append(v)
    h1 = jnp.concatenate(hs, axis=1)
    h1 = jnp.where(h1 > 0, h1, jnp.exp(jnp.minimum(h1, 0.0)) - 1.0)
    t2 = jnp.dot(h1, w2_ref[...], preferred_element_type=_f32)
    t3 = jnp.dot(h1, w3_ref[...], preferred_element_type=_f32)
    pad = jnp.zeros((t2.shape[0], _LANE), _f32)
    tab_ref[0] = jnp.concatenate([t2, pad], axis=1)
    tab_ref[1] = jnp.concatenate([t3, pad], axis=1)
    ab_ref[:, 0:1] = jnp.sum(t2 * as_ref[0][None, :], 1, keepdims=True)
    ab_ref[:, 1:2] = jnp.sum(t3 * as_ref[1][None, :], 1, keepdims=True)
    ab_ref[:, 2:3] = jnp.sum(t2 * ad_ref[0][None, :], 1, keepdims=True)
    ab_ref[:, 3:4] = jnp.sum(t3 * ad_ref[1][None, :], 1, keepdims=True)


def _proj23(num1, b1, W2, W3, as2, ad2, as3, ad3):
    R = 2048
    CP = num1.shape[-1]
    D = _HEADS * 2 * _OUT
    as23 = jnp.concatenate([as2, as3], axis=0)
    ad23 = jnp.concatenate([ad2, ad3], axis=0)
    return pl.pallas_call(
        _proj23_body,
        grid=(_NP // R,),
        in_specs=[
            pl.BlockSpec((_HEADS, R, CP), lambda i: (0, i, 0)),
            pl.BlockSpec((1, D), lambda i: (0, 0)),
            pl.BlockSpec((D, _OUT), lambda i: (0, 0)),
            pl.BlockSpec((D, _OUT), lambda i: (0, 0)),
            pl.BlockSpec((2, _OUT), lambda i: (0, 0)),
            pl.BlockSpec((2, _OUT), lambda i: (0, 0)),
        ],
        out_specs=[
            pl.BlockSpec((2, R, _OUT + _LANE), lambda i: (0, i, 0)),
            pl.BlockSpec((R, 4), lambda i: (i, 0)),
        ],
        out_shape=[
            jax.ShapeDtypeStruct((2, _NP, _OUT + _LANE), _f32),
            jax.ShapeDtypeStruct((_NP, 4), _f32),
        ],
    )(num1, b1.reshape(1, D), W2, W3, as23, ad23)


def _fin23_body(num_ref, b_ref, out_ref):
    for hd in range(2):
        blk = num_ref[hd]
        out_ref[hd] = blk[:, :_OUT] / (blk[:, _OUT:_OUT + 1] + 1e-16) \
            + b_ref[hd][None, :]


def _fin23(num23, b23):
    R = 2048
    CP = num23.shape[-1]
    return pl.pallas_call(
        _fin23_body,
        grid=(_NP // R,),
        in_specs=[
            pl.BlockSpec((2, R, CP), lambda i: (0, i, 0)),
            pl.BlockSpec((2, _OUT), lambda i: (0, 0)),
        ],
        out_specs=pl.BlockSpec((2, R, _OUT), lambda i: (0, i, 0)),
        out_shape=jax.ShapeDtypeStruct((2, _NP, _OUT), _f32),
    )(num23, b23)


def _wts_body(nh, ba, per_tile, abt_ref, src_ref, dst_ref, w_ref,
              sidx, didx, abuf, wbuf, sem):
    core = lax.axis_index("c")
    sub = lax.axis_index("s")
    wid = sub * 2 + core
    nchunks = per_tile // ba

    def chunk(i, _):
        base = wid * per_tile + i * ba
        pltpu.sync_copy(src_ref.at[pl.ds(base, ba)], sidx)
        pltpu.sync_copy(dst_ref.at[pl.ds(base, ba)], didx)
        cps = []
        for h in range(nh):
            cps.append(pltpu.async_copy(
                abt_ref.at[h].at[sidx],
                abuf.at[pl.ds(h * ba, ba)], sem))
            cps.append(pltpu.async_copy(
                abt_ref.at[nh + h].at[didx],
                abuf.at[pl.ds((nh + h) * ba, ba)], sem))
        for cp in cps:
            cp.wait()

        def grp(g, _):
            rows = g * _LANE + _iota16()
            for h in range(nh):
                sa = plsc.load_gather(abuf, [h * ba + rows])
                da = plsc.load_gather(abuf, [(nh + h) * ba + rows])
                al = sa + da
                al = jnp.maximum(al, 0.2 * al)
                plsc.store_scatter(wbuf, [h * ba + rows], jnp.exp(al))
            return 0

        lax.fori_loop(0, ba // _LANE, grp, 0)
        for h in range(nh):
            pltpu.sync_copy(wbuf.at[pl.ds(h * ba, ba)],
                            w_ref.at[h].at[pl.ds(base, ba)])
        return 0

    lax.fori_loop(0, nchunks, chunk, 0)


def _wts(abt, src, dst, nh):
    e = src.shape[0]
    per_tile = e // 32
    ba = min(128, per_tile)
    assert per_tile % ba == 0 and ba % _LANE == 0 and ba % 128 == 0
    mesh = plsc.VectorSubcoreMesh(core_axis_name="c", subcore_axis_name="s")
    body = functools.partial(_wts_body, nh, ba, per_tile)
    return pl.kernel(
        body,
        out_type=jax.ShapeDtypeStruct((nh, e), _f32),
        mesh=mesh,
        compiler_params=pltpu.CompilerParams(use_tc_tiling_on_sc=False, needs_layout_passes=False),
        scratch_types=[
            pltpu.VMEM((ba,), _i32),
            pltpu.VMEM((ba,), _i32),
            pltpu.VMEM((2 * nh * ba,), _f32),
            pltpu.VMEM((nh * ba,), _f32),
            pltpu.SemaphoreType.DMA,
        ],
    )(abt, src, dst)


def _agg_body(nh, c, cp, b, per_tile, rows_tile,
              tab_ref, w_ref, src_ref, dst_ref, z_ref, num_ref,
              acc, rbuf0, rbuf1, sidx0, sidx1, didx0, didx1, wch0, wch1,
              semg0, semg1, sems0, sems1):
    core = lax.axis_index("c")
    sub = lax.axis_index("s")
    hpc = nh // 2
    nchunks = per_tile // b
    rbufs = (rbuf0, rbuf1)
    sidxs = (sidx0, sidx1)
    didxs = (didx0, didx1)
    wchs = (wch0, wch1)
    semgs = (semg0, semg1)
    semss = (sems0, sems1)
    lane0 = (_iota16() == 0).astype(_f32)

    for k in range(hpc):
        h = core * hpc + k
        pltpu.sync_copy(z_ref.at[pl.ds(sub * rows_tile, rows_tile)],
                        acc.at[pl.ds(sub * rows_tile, rows_tile)])
        plsc.subcore_barrier()

        def stage_a(i, s):
            base = sub * per_tile + i * b
            pltpu.sync_copy(src_ref.at[pl.ds(base, b)], sidxs[s])
            pltpu.sync_copy(dst_ref.at[pl.ds(base, b)], didxs[s])
            pltpu.sync_copy(w_ref.at[h].at[pl.ds(base, b)], wchs[s])
            pltpu.async_copy(tab_ref.at[h].at[sidxs[s]], rbufs[s], semgs[s])

        def wait_scat(s):
            pltpu.make_async_copy(rbufs[s], acc.at[didxs[s]],
                                  semss[s]).wait()

        def stage_b(s):
            pltpu.make_async_copy(tab_ref.at[h].at[sidxs[s]], rbufs[s],
                                  semgs[s]).wait()

            def grp(g, _):
                for e16 in range(_LANE):
                    e2 = g * _LANE + e16
                    wv = plsc.load_gather(wchs[s], [_full16(e2)])
                    for j in range(c // _LANE):
                        sl = pl.ds(j * _LANE, _LANE)
                        rbufs[s][e2, sl] = rbufs[s][e2, sl] * wv
                    rbufs[s][e2, pl.ds(c, _LANE)] = wv * lane0
                return 0

            lax.fori_loop(0, b // _LANE, grp, 0)
            pltpu.async_copy(rbufs[s], acc.at[didxs[s]], semss[s], add=True)

        def it(i2, _):
            a = 2 * i2

            @pl.when(i2 > 0)
            def _():
                wait_scat(0)

            stage_a(a, 0)

            @pl.when(i2 > 0)
            def _():
                stage_b(1)
                wait_scat(1)

            stage_a(a + 1, 1)
            stage_b(0)
            return 0

        lax.fori_loop(0, nchunks // 2, it, 0)
        stage_b(1)
        wait_scat(0)
        wait_scat(1)
        plsc.subcore_barrier()
        pltpu.sync_copy(acc.at[pl.ds(sub * rows_tile, rows_tile)],
                        num_ref.at[h].at[pl.ds(sub * rows_tile, rows_tile)])
        plsc.subcore_barrier()


def _agg(tab, w, src, dst, c):
    nh = tab.shape[0]
    e = src.shape[0]
    cp = c + _LANE
    assert tab.shape[2] == cp
    per_tile = e // 16
    b = min(128, per_tile)
    rows_tile = _NP // 16
    assert per_tile % b == 0 and (per_tile // b) % 2 == 0 and b % 128 == 0
    mesh = plsc.VectorSubcoreMesh(core_axis_name="c", subcore_axis_name="s")
    z = jnp.zeros((_NP, cp), _f32)
    body = functools.partial(_agg_body, nh, c, cp, b, per_tile, rows_tile)
    return pl.kernel(
        body,
        out_type=jax.ShapeDtypeStruct((nh, _NP, cp), _f32),
        mesh=mesh,
        compiler_params=pltpu.CompilerParams(use_tc_tiling_on_sc=False, needs_layout_passes=False),
        scratch_types=[
            pltpu.VMEM_SHARED((_NP, cp), _f32),
            pltpu.VMEM((b, cp), _f32),
            pltpu.VMEM((b, cp), _f32),
            pltpu.VMEM((b,), _i32),
            pltpu.VMEM((b,), _i32),
            pltpu.VMEM((b,), _i32),
            pltpu.VMEM((b,), _i32),
            pltpu.VMEM((b,), _f32),
            pltpu.VMEM((b,), _f32),
            pltpu.SemaphoreType.DMA,
            pltpu.SemaphoreType.DMA,
            pltpu.SemaphoreType.DMA,
            pltpu.SemaphoreType.DMA,
        ],
    )(tab, w, src, dst, z)


def kernel(x, edge_index, W1, as1, ad1, b1, W2, as2, ad2, b2,
           W3, as3, ad3, b3):
    src = edge_index[0].astype(_i32)
    dst = edge_index[1].astype(_i32)
    pad = (jnp.arange(_EP - _E, dtype=_i32) % 192) + (_NP - 192)
    src = jnp.concatenate([src, pad])
    dst = jnp.concatenate([dst, pad])
    xp = jnp.zeros((_NP, _IN), _f32).at[:_N].set(x)

    tab1, ab1 = _proj1(xp, W1, as1, ad1)
    w1 = _wts(ab1.T, src, dst, _HEADS)
    num1 = _agg(tab1, w1, src, dst, 128)

    tab23, ab23 = _proj23(num1, b1, W2, W3, as2, ad2, as3, ad3)
    w23 = _wts(ab23.T, src, dst, 2)
    num23 = _agg(tab23, w23, src, dst, _OUT)

    out = _fin23(num23, jnp.stack([b2, b3]))
    return out[0, :_N], out[1, :_N]


# speedup vs baseline: 22.2630x; 1.2071x over previous
import functools
import jax
import jax.numpy as jnp
from jax import lax
from jax.experimental import pallas as pl
from jax.experimental.pallas import tpu as pltpu
from jax.experimental.pallas import tpu_sc as plsc

_N = 10000
_E = 320000
_IN = 128
_OUT = 64
_HEADS = 4
_NP = 10240
_EP = 327680
_LANE = 16

_f32 = jnp.float32
_i32 = jnp.int32


def _iota16():
    return lax.iota(_i32, _LANE)


def _full16(v):
    return jnp.full((_LANE,), v, _i32)


def _proj1_body(x_ref, w_ref, as_ref, ad_ref, tab_ref, ab_ref):
    h = jnp.dot(x_ref[...], w_ref[...], preferred_element_type=_f32)
    pad = jnp.zeros((h.shape[0], _LANE), _f32)
    for hd in range(_HEADS):
        hh = h[:, hd * 128:(hd + 1) * 128]
        tab_ref[hd] = jnp.concatenate([hh, pad], axis=1)
        ab_ref[:, hd:hd + 1] = jnp.sum(hh * as_ref[hd][None, :], axis=1,
                                       keepdims=True)
        ab_ref[:, _HEADS + hd:_HEADS + hd + 1] = jnp.sum(
            hh * ad_ref[hd][None, :], axis=1, keepdims=True)


def _proj1(xp, W1, as1, ad1):
    R = 2048
    grid = (_NP // R,)
    return pl.pallas_call(
        _proj1_body,
        grid=grid,
        in_specs=[
            pl.BlockSpec((R, _IN), lambda i: (i, 0)),
            pl.BlockSpec((_IN, _HEADS * 128), lambda i: (0, 0)),
            pl.BlockSpec((_HEADS, 128), lambda i: (0, 0)),
            pl.BlockSpec((_HEADS, 128), lambda i: (0, 0)),
        ],
        out_specs=[
            pl.BlockSpec((_HEADS, R, 128 + _LANE), lambda i: (0, i, 0)),
            pl.BlockSpec((R, 2 * _HEADS), lambda i: (i, 0)),
        ],
        out_shape=[
            jax.ShapeDtypeStruct((_HEADS, _NP, 128 + _LANE), _f32),
            jax.ShapeDtypeStruct((_NP, 2 * _HEADS), _f32),
        ],
    )(xp, W1, as1, ad1)


def _proj23_body(num_ref, b1_ref, w2_ref, w3_ref, as_ref, ad_ref,
                 tab_ref, ab_ref):
    hs = []
    for hd in range(_HEADS):
        blk = num_ref[hd]
        v = blk[:, :128] / (blk[:, 128:129] + 1e-16)
        v = v + b1_ref[0, hd * 128:(hd + 1) * 128][None, :]
        hs.append(v)
    h1 = jnp.concatenate(hs, axis=1)
    h1 = jnp.where(h1 > 0, h1, jnp.exp(jnp.minimum(h1, 0.0)) - 1.0)
    t2 = jnp.dot(h1, w2_ref[...], preferred_element_type=_f32)
    t3 = jnp.dot(h1, w3_ref[...], preferred_element_type=_f32)
    pad = jnp.zeros((t2.shape[0], _LANE), _f32)
    tab_ref[0] = jnp.concatenate([t2, pad], axis=1)
    tab_ref[1] = jnp.concatenate([t3, pad], axis=1)
    ab_ref[:, 0:1] = jnp.sum(t2 * as_ref[0][None, :], 1, keepdims=True)
    ab_ref[:, 1:2] = jnp.sum(t3 * as_ref[1][None, :], 1, keepdims=True)
    ab_ref[:, 2:3] = jnp.sum(t2 * ad_ref[0][None, :], 1, keepdims=True)
    ab_ref[:, 3:4] = jnp.sum(t3 * ad_ref[1][None, :], 1, keepdims=True)


def _proj23(num1, b1, W2, W3, as2, ad2, as3, ad3):
    R = 2048
    CP = num1.shape[-1]
    D = _HEADS * 2 * _OUT
    as23 = jnp.concatenate([as2, as3], axis=0)
    ad23 = jnp.concatenate([ad2, ad3], axis=0)
    return pl.pallas_call(
        _proj23_body,
        grid=(_NP // R,),
        in_specs=[
            pl.BlockSpec((_HEADS, R, CP), lambda i: (0, i, 0)),
            pl.BlockSpec((1, D), lambda i: (0, 0)),
            pl.BlockSpec((D, _OUT), lambda i: (0, 0)),
            pl.BlockSpec((D, _OUT), lambda i: (0, 0)),
            pl.BlockSpec((2, _OUT), lambda i: (0, 0)),
            pl.BlockSpec((2, _OUT), lambda i: (0, 0)),
        ],
        out_specs=[
            pl.BlockSpec((2, R, _OUT + _LANE), lambda i: (0, i, 0)),
            pl.BlockSpec((R, 4), lambda i: (i, 0)),
        ],
        out_shape=[
            jax.ShapeDtypeStruct((2, _NP, _OUT + _LANE), _f32),
            jax.ShapeDtypeStruct((_NP, 4), _f32),
        ],
    )(num1, b1.reshape(1, D), W2, W3, as23, ad23)


def _fin23_body(num_ref, b_ref, out_ref):
    for hd in range(2):
        blk = num_ref[hd]
        out_ref[hd] = blk[:, :_OUT] / (blk[:, _OUT:_OUT + 1] + 1e-16) \
            + b_ref[hd][None, :]


def _fin23(num23, b23):
    R = 2048
    CP = num23.shape[-1]
    return pl.pallas_call(
        _fin23_body,
        grid=(_NP // R,),
        in_specs=[
            pl.BlockSpec((2, R, CP), lambda i: (0, i, 0)),
            pl.BlockSpec((2, _OUT), lambda i: (0, 0)),
        ],
        out_specs=pl.BlockSpec((2, R, _OUT), lambda i: (0, i, 0)),
        out_shape=jax.ShapeDtypeStruct((2, _NP, _OUT), _f32),
    )(num23, b23)


def _wts_body(nh, ba, per_tile, abt_ref, src_ref, dst_ref, w_ref,
              sidx, didx, abuf, wbuf, sem):
    core = lax.axis_index("c")
    sub = lax.axis_index("s")
    wid = sub * 2 + core
    nchunks = per_tile // ba

    def chunk(i, _):
        base = wid * per_tile + i * ba
        pltpu.sync_copy(src_ref.at[pl.ds(base, ba)], sidx)
        pltpu.sync_copy(dst_ref.at[pl.ds(base, ba)], didx)
        cps = []
        for h in range(nh):
            cps.append(pltpu.async_copy(
                abt_ref.at[h].at[sidx],
                abuf.at[pl.ds(h * ba, ba)], sem))
            cps.append(pltpu.async_copy(
                abt_ref.at[nh + h].at[didx],
                abuf.at[pl.ds((nh + h) * ba, ba)], sem))
        for cp in cps:
            cp.wait()

        def grp(g, _):
            rows = g * _LANE + _iota16()
            for h in range(nh):
                sa = plsc.load_gather(abuf, [h * ba + rows])
                da = plsc.load_gather(abuf, [(nh + h) * ba + rows])
                al = sa + da
                al = jnp.maximum(al, 0.2 * al)
                plsc.store_scatter(wbuf, [h * ba + rows], jnp.exp(al))
            return 0

        lax.fori_loop(0, ba // _LANE, grp, 0)
        for h in range(nh):
            pltpu.sync_copy(wbuf.at[pl.ds(h * ba, ba)],
                            w_ref.at[h].at[pl.ds(base, ba)])
        return 0

    lax.fori_loop(0, nchunks, chunk, 0)


def _wts(abt, src, dst, nh):
    e = src.shape[0]
    per_tile = e // 32
    ba = min(128, per_tile)
    assert per_tile % ba == 0 and ba % _LANE == 0 and ba % 128 == 0
    mesh = plsc.VectorSubcoreMesh(core_axis_name="c", subcore_axis_name="s")
    body = functools.partial(_wts_body, nh, ba, per_tile)
    return pl.kernel(
        body,
        out_type=jax.ShapeDtypeStruct((nh, e), _f32),
        mesh=mesh,
        compiler_params=pltpu.CompilerParams(use_tc_tiling_on_sc=False, needs_layout_passes=False),
        scratch_types=[
            pltpu.VMEM((ba,), _i32),
            pltpu.VMEM((ba,), _i32),
            pltpu.VMEM((2 * nh * ba,), _f32),
            pltpu.VMEM((nh * ba,), _f32),
            pltpu.SemaphoreType.DMA,
        ],
    )(abt, src, dst)


def _agg_body(nh, c, cp, b, per_tile, rows_tile,
              tab_ref, w_ref, src_ref, dst_ref, z_ref, num_ref,
              acc, rbuf0, rbuf1, sidx0, sidx1, didx0, didx1, wch0, wch1,
              semg0, semg1, sems0, sems1, semi):
    core = lax.axis_index("c")
    sub = lax.axis_index("s")
    hpc = nh // 2
    nchunks = per_tile // b
    rbufs = (rbuf0, rbuf1)
    sidxs = (sidx0, sidx1)
    didxs = (didx0, didx1)
    wchs = (wch0, wch1)
    semgs = (semg0, semg1)
    semss = (sems0, sems1)
    lane0 = (_iota16() == 0).astype(_f32)

    for k in range(hpc):
        h = core * hpc + k
        pltpu.sync_copy(z_ref.at[pl.ds(sub * rows_tile, rows_tile)],
                        acc.at[pl.ds(sub * rows_tile, rows_tile)])
        plsc.subcore_barrier()

        def stage_a(i, s):
            base = sub * per_tile + i * b
            c1 = pltpu.async_copy(src_ref.at[pl.ds(base, b)], sidxs[s], semi)
            c2 = pltpu.async_copy(dst_ref.at[pl.ds(base, b)], didxs[s], semi)
            c3 = pltpu.async_copy(w_ref.at[h].at[pl.ds(base, b)],
                                  wchs[s], semi)
            c1.wait()
            c2.wait()
            c3.wait()
            pltpu.async_copy(tab_ref.at[h].at[sidxs[s]], rbufs[s], semgs[s])

        def wait_scat(s):
            pltpu.make_async_copy(rbufs[s], acc.at[didxs[s]],
                                  semss[s]).wait()

        def stage_b(s):
            pltpu.make_async_copy(tab_ref.at[h].at[sidxs[s]], rbufs[s],
                                  semgs[s]).wait()

            def grp(g, _):
                for e16 in range(_LANE):
                    e2 = g * _LANE + e16
                    wv = plsc.load_gather(wchs[s], [_full16(e2)])
                    for j in range(c // _LANE):
                        sl = pl.ds(j * _LANE, _LANE)
                        rbufs[s][e2, sl] = rbufs[s][e2, sl] * wv
                    rbufs[s][e2, pl.ds(c, _LANE)] = wv * lane0
                return 0

            lax.fori_loop(0, b // _LANE, grp, 0)
            pltpu.async_copy(rbufs[s], acc.at[didxs[s]], semss[s], add=True)

        def it(i2, _):
            a = 2 * i2

            @pl.when(i2 > 0)
            def _():
                wait_scat(0)

            stage_a(a, 0)

            @pl.when(i2 > 0)
            def _():
                stage_b(1)
                wait_scat(1)

            stage_a(a + 1, 1)
            stage_b(0)
            return 0

        lax.fori_loop(0, nchunks // 2, it, 0)
        stage_b(1)
        wait_scat(0)
        wait_scat(1)
        plsc.subcore_barrier()
        pltpu.sync_copy(acc.at[pl.ds(sub * rows_tile, rows_tile)],
                        num_ref.at[h].at[pl.ds(sub * rows_tile, rows_tile)])
        plsc.subcore_barrier()


def _agg(tab, w, src, dst, c):
    nh = tab.shape[0]
    e = src.shape[0]
    cp = c + _LANE
    assert tab.shape[2] == cp
    per_tile = e // 16
    b = min(128, per_tile)
    rows_tile = _NP // 16
    assert per_tile % b == 0 and (per_tile // b) % 2 == 0 and b % 128 == 0
    mesh = plsc.VectorSubcoreMesh(core_axis_name="c", subcore_axis_name="s")
    z = jnp.zeros((_NP, cp), _f32)
    body = functools.partial(_agg_body, nh, c, cp, b, per_tile, rows_tile)
    return pl.kernel(
        body,
        out_type=jax.ShapeDtypeStruct((nh, _NP, cp), _f32),
        mesh=mesh,
        compiler_params=pltpu.CompilerParams(use_tc_tiling_on_sc=False, needs_layout_passes=False),
        scratch_types=[
            pltpu.VMEM_SHARED((_NP, cp), _f32),
            pltpu.VMEM((b, cp), _f32),
            pltpu.VMEM((b, cp), _f32),
            pltpu.VMEM((b,), _i32),
            pltpu.VMEM((b,), _i32),
            pltpu.VMEM((b,), _i32),
            pltpu.VMEM((b,), _i32),
            pltpu.VMEM((b,), _f32),
            pltpu.VMEM((b,), _f32),
            pltpu.SemaphoreType.DMA,
            pltpu.SemaphoreType.DMA,
            pltpu.SemaphoreType.DMA,
            pltpu.SemaphoreType.DMA,
            pltpu.SemaphoreType.DMA,
        ],
    )(tab, w, src, dst, z)


def kernel(x, edge_index, W1, as1, ad1, b1, W2, as2, ad2, b2,
           W3, as3, ad3, b3):
    src = edge_index[0].astype(_i32)
    dst = edge_index[1].astype(_i32)
    pad = (jnp.arange(_EP - _E, dtype=_i32) % 192) + (_NP - 192)
    src = jnp.concatenate([src, pad])
    dst = jnp.concatenate([dst, pad])
    xp = jnp.zeros((_NP, _IN), _f32).at[:_N].set(x)

    tab1, ab1 = _proj1(xp, W1, as1, ad1)
    w1 = _wts(ab1.T, src, dst, _HEADS)
    num1 = _agg(tab1, w1, src, dst, 128)

    tab23, ab23 = _proj23(num1, b1, W2, W3, as2, ad2, as3, ad3)
    w23 = _wts(ab23.T, src, dst, 2)
    num23 = _agg(tab23, w23, src, dst, _OUT)

    out = _fin23(num23, jnp.stack([b2, b3]))
    return out[0, :_N], out[1, :_N]


# speedup vs baseline: 22.9996x; 1.0331x over previous
import functools
import jax
import jax.numpy as jnp
from jax import lax
from jax.experimental import pallas as pl
from jax.experimental.pallas import tpu as pltpu
from jax.experimental.pallas import tpu_sc as plsc

_N = 10000
_E = 320000
_IN = 128
_OUT = 64
_HEADS = 4
_NP = 10240
_EP = 327680
_LANE = 16

_f32 = jnp.float32
_i32 = jnp.int32


def _iota16():
    return lax.iota(_i32, _LANE)


def _full16(v):
    return jnp.full((_LANE,), v, _i32)


def _proj1_body(x_ref, w_ref, as_ref, ad_ref, tab_ref, ab_ref):
    h = jnp.dot(x_ref[...], w_ref[...], preferred_element_type=_f32)
    pad = jnp.zeros((h.shape[0], _LANE), _f32)
    for hd in range(_HEADS):
        hh = h[:, hd * 128:(hd + 1) * 128]
        tab_ref[hd] = jnp.concatenate([hh, pad], axis=1)
        ab_ref[:, hd:hd + 1] = jnp.sum(hh * as_ref[hd][None, :], axis=1,
                                       keepdims=True)
        ab_ref[:, _HEADS + hd:_HEADS + hd + 1] = jnp.sum(
            hh * ad_ref[hd][None, :], axis=1, keepdims=True)


def _proj1(xp, W1, as1, ad1):
    R = 2048
    grid = (_NP // R,)
    return pl.pallas_call(
        _proj1_body,
        grid=grid,
        in_specs=[
            pl.BlockSpec((R, _IN), lambda i: (i, 0)),
            pl.BlockSpec((_IN, _HEADS * 128), lambda i: (0, 0)),
            pl.BlockSpec((_HEADS, 128), lambda i: (0, 0)),
            pl.BlockSpec((_HEADS, 128), lambda i: (0, 0)),
        ],
        out_specs=[
            pl.BlockSpec((_HEADS, R, 128 + _LANE), lambda i: (0, i, 0)),
            pl.BlockSpec((R, 2 * _HEADS), lambda i: (i, 0)),
        ],
        out_shape=[
            jax.ShapeDtypeStruct((_HEADS, _NP, 128 + _LANE), _f32),
            jax.ShapeDtypeStruct((_NP, 2 * _HEADS), _f32),
        ],
    )(xp, W1, as1, ad1)


def _proj23_body(num_ref, b1_ref, w2_ref, w3_ref, as_ref, ad_ref,
                 tab_ref, ab_ref):
    hs = []
    for hd in range(_HEADS):
        blk = num_ref[hd]
        v = blk[:, :128] / (blk[:, 128:129] + 1e-16)
        v = v + b1_ref[0, hd * 128:(hd + 1) * 128][None, :]
        hs.append(v)
    h1 = jnp.concatenate(hs, axis=1)
    h1 = jnp.where(h1 > 0, h1, jnp.exp(jnp.minimum(h1, 0.0)) - 1.0)
    t2 = jnp.dot(h1, w2_ref[...], preferred_element_type=_f32)
    t3 = jnp.dot(h1, w3_ref[...], preferred_element_type=_f32)
    pad = jnp.zeros((t2.shape[0], _LANE), _f32)
    tab_ref[0] = jnp.concatenate([t2, pad], axis=1)
    tab_ref[1] = jnp.concatenate([t3, pad], axis=1)
    ab_ref[:, 0:1] = jnp.sum(t2 * as_ref[0][None, :], 1, keepdims=True)
    ab_ref[:, 1:2] = jnp.sum(t3 * as_ref[1][None, :], 1, keepdims=True)
    ab_ref[:, 2:3] = jnp.sum(t2 * ad_ref[0][None, :], 1, keepdims=True)
    ab_ref[:, 3:4] = jnp.sum(t3 * ad_ref[1][None, :], 1, keepdims=True)


def _proj23(num1, b1, W2, W3, as2, ad2, as3, ad3):
    R = 2048
    CP = num1.shape[-1]
    D = _HEADS * 2 * _OUT
    as23 = jnp.concatenate([as2, as3], axis=0)
    ad23 = jnp.concatenate([ad2, ad3], axis=0)
    return pl.pallas_call(
        _proj23_body,
        grid=(_NP // R,),
        in_specs=[
            pl.BlockSpec((_HEADS, R, CP), lambda i: (0, i, 0)),
            pl.BlockSpec((1, D), lambda i: (0, 0)),
            pl.BlockSpec((D, _OUT), lambda i: (0, 0)),
            pl.BlockSpec((D, _OUT), lambda i: (0, 0)),
            pl.BlockSpec((2, _OUT), lambda i: (0, 0)),
            pl.BlockSpec((2, _OUT), lambda i: (0, 0)),
        ],
        out_specs=[
            pl.BlockSpec((2, R, _OUT + _LANE), lambda i: (0, i, 0)),
            pl.BlockSpec((R, 4), lambda i: (i, 0)),
        ],
        out_shape=[
            jax.ShapeDtypeStruct((2, _NP, _OUT + _LANE), _f32),
            jax.ShapeDtypeStruct((_NP, 4), _f32),
        ],
    )(num1, b1.reshape(1, D), W2, W3, as23, ad23)


def _fin23_body(num_ref, b_ref, out_ref):
    for hd in range(2):
        blk = num_ref[hd]
        out_ref[hd] = blk[:, :_OUT] / (blk[:, _OUT:_OUT + 1] + 1e-16) \
            + b_ref[hd][None, :]


def _fin23(num23, b23):
    R = 2048
    CP = num23.shape[-1]
    return pl.pallas_call(
        _fin23_body,
        grid=(_NP // R,),
        in_specs=[
            pl.BlockSpec((2, R, CP), lambda i: (0, i, 0)),
            pl.BlockSpec((2, _OUT), lambda i: (0, 0)),
        ],
        out_specs=pl.BlockSpec((2, R, _OUT), lambda i: (0, i, 0)),
        out_shape=jax.ShapeDtypeStruct((2, _NP, _OUT), _f32),
    )(num23, b23)


def _wts_body(nh, ba, per_tile, abt_ref, src_ref, dst_ref, w_ref,
              sidx, didx, abuf, wbuf, sem):
    core = lax.axis_index("c")
    sub = lax.axis_index("s")
    wid = sub * 2 + core
    nchunks = per_tile // ba

    def chunk(i, _):
        base = wid * per_tile + i * ba
        ci1 = pltpu.async_copy(src_ref.at[pl.ds(base, ba)], sidx, sem)
        ci2 = pltpu.async_copy(dst_ref.at[pl.ds(base, ba)], didx, sem)
        ci1.wait()
        ci2.wait()
        cps = []
        for h in range(nh):
            cps.append(pltpu.async_copy(
                abt_ref.at[h].at[sidx],
                abuf.at[pl.ds(h * ba, ba)], sem))
            cps.append(pltpu.async_copy(
                abt_ref.at[nh + h].at[didx],
                abuf.at[pl.ds((nh + h) * ba, ba)], sem))
        for cp in cps:
            cp.wait()

        def grp(g, _):
            rows = g * _LANE + _iota16()
            for h in range(nh):
                sa = plsc.load_gather(abuf, [h * ba + rows])
                da = plsc.load_gather(abuf, [(nh + h) * ba + rows])
                al = sa + da
                al = jnp.maximum(al, 0.2 * al)
                plsc.store_scatter(wbuf, [h * ba + rows], jnp.exp(al))
            return 0

        lax.fori_loop(0, ba // _LANE, grp, 0)
        for h in range(nh):
            pltpu.sync_copy(wbuf.at[pl.ds(h * ba, ba)],
                            w_ref.at[h].at[pl.ds(base, ba)])
        return 0

    lax.fori_loop(0, nchunks, chunk, 0)


def _wts(abt, src, dst, nh):
    e = src.shape[0]
    per_tile = e // 32
    ba = min(128, per_tile)
    assert per_tile % ba == 0 and ba % _LANE == 0 and ba % 128 == 0
    mesh = plsc.VectorSubcoreMesh(core_axis_name="c", subcore_axis_name="s")
    body = functools.partial(_wts_body, nh, ba, per_tile)
    return pl.kernel(
        body,
        out_type=jax.ShapeDtypeStruct((nh, e), _f32),
        mesh=mesh,
        compiler_params=pltpu.CompilerParams(use_tc_tiling_on_sc=False, needs_layout_passes=False),
        scratch_types=[
            pltpu.VMEM((ba,), _i32),
            pltpu.VMEM((ba,), _i32),
            pltpu.VMEM((2 * nh * ba,), _f32),
            pltpu.VMEM((nh * ba,), _f32),
            pltpu.SemaphoreType.DMA,
        ],
    )(abt, src, dst)


def _agg_body(nh, c, cp, b, per_tile, rows_tile,
              tab_ref, w_ref, src_ref, dst_ref, z_ref, num_ref,
              acc, rbuf0, rbuf1, sidx0, sidx1, didx0, didx1, wch0, wch1,
              semg0, semg1, sems0, sems1, semi):
    core = lax.axis_index("c")
    sub = lax.axis_index("s")
    hpc = nh // 2
    nchunks = per_tile // b
    rbufs = (rbuf0, rbuf1)
    sidxs = (sidx0, sidx1)
    didxs = (didx0, didx1)
    wchs = (wch0, wch1)
    semgs = (semg0, semg1)
    semss = (sems0, sems1)
    lane0 = (_iota16() == 0).astype(_f32)

    for k in range(hpc):
        h = core * hpc + k
        pltpu.sync_copy(z_ref.at[pl.ds(sub * rows_tile, rows_tile)],
                        acc.at[pl.ds(sub * rows_tile, rows_tile)])
        plsc.subcore_barrier()

        def stage_a(i, s):
            base = sub * per_tile + i * b
            c1 = pltpu.async_copy(src_ref.at[pl.ds(base, b)], sidxs[s], semi)
            c2 = pltpu.async_copy(dst_ref.at[pl.ds(base, b)], didxs[s], semi)
            c3 = pltpu.async_copy(w_ref.at[h].at[pl.ds(base, b)],
                                  wchs[s], semi)
            c1.wait()
            c2.wait()
            c3.wait()
            pltpu.async_copy(tab_ref.at[h].at[sidxs[s]], rbufs[s], semgs[s])

        def wait_scat(s):
            pltpu.make_async_copy(rbufs[s], acc.at[didxs[s]],
                                  semss[s]).wait()

        def stage_b(s):
            pltpu.make_async_copy(tab_ref.at[h].at[sidxs[s]], rbufs[s],
                                  semgs[s]).wait()

            def grp(g, _):
                for e16 in range(_LANE):
                    e2 = g * _LANE + e16
                    wv = plsc.load_gather(wchs[s], [_full16(e2)])
                    for j in range(c // _LANE):
                        sl = pl.ds(j * _LANE, _LANE)
                        rbufs[s][e2, sl] = rbufs[s][e2, sl] * wv
                    rbufs[s][e2, pl.ds(c, _LANE)] = wv * lane0
                return 0

            lax.fori_loop(0, b // _LANE, grp, 0)
            pltpu.async_copy(rbufs[s], acc.at[didxs[s]], semss[s], add=True)

        def it(i2, _):
            a = 2 * i2

            @pl.when(i2 > 0)
            def _():
                wait_scat(0)

            stage_a(a, 0)

            @pl.when(i2 > 0)
            def _():
                stage_b(1)
                wait_scat(1)

            stage_a(a + 1, 1)
            stage_b(0)
            return 0

        lax.fori_loop(0, nchunks // 2, it, 0)
        stage_b(1)
        wait_scat(0)
        wait_scat(1)
        plsc.subcore_barrier()
        pltpu.sync_copy(acc.at[pl.ds(sub * rows_tile, rows_tile)],
                        num_ref.at[h].at[pl.ds(sub * rows_tile, rows_tile)])
        plsc.subcore_barrier()


def _agg(tab, w, src, dst, c):
    nh = tab.shape[0]
    e = src.shape[0]
    cp = c + _LANE
    assert tab.shape[2] == cp
    per_tile = e // 16
    b = min(128, per_tile)
    rows_tile = _NP // 16
    assert per_tile % b == 0 and (per_tile // b) % 2 == 0 and b % 128 == 0
    mesh = plsc.VectorSubcoreMesh(core_axis_name="c", subcore_axis_name="s")
    z = jnp.zeros((_NP, cp), _f32)
    body = functools.partial(_agg_body, nh, c, cp, b, per_tile, rows_tile)
    return pl.kernel(
        body,
        out_type=jax.ShapeDtypeStruct((nh, _NP, cp), _f32),
        mesh=mesh,
        compiler_params=pltpu.CompilerParams(use_tc_tiling_on_sc=False, needs_layout_passes=False),
        scratch_types=[
            pltpu.VMEM_SHARED((_NP, cp), _f32),
            pltpu.VMEM((b, cp), _f32),
            pltpu.VMEM((b, cp), _f32),
            pltpu.VMEM((b,), _i32),
            pltpu.VMEM((b,), _i32),
            pltpu.VMEM((b,), _i32),
            pltpu.VMEM((b,), _i32),
            pltpu.VMEM((b,), _f32),
            pltpu.VMEM((b,), _f32),
            pltpu.SemaphoreType.DMA,
            pltpu.SemaphoreType.DMA,
            pltpu.SemaphoreType.DMA,
            pltpu.SemaphoreType.DMA,
            pltpu.SemaphoreType.DMA,
        ],
    )(tab, w, src, dst, z)


def kernel(x, edge_index, W1, as1, ad1, b1, W2, as2, ad2, b2,
           W3, as3, ad3, b3):
    src = edge_index[0].astype(_i32)
    dst = edge_index[1].astype(_i32)
    pad = (jnp.arange(_EP - _E, dtype=_i32) % 192) + (_NP - 192)
    src = jnp.concatenate([src, pad])
    dst = jnp.concatenate([dst, pad])
    xp = jnp.zeros((_NP, _IN), _f32).at[:_N].set(x)

    tab1, ab1 = _proj1(xp, W1, as1, ad1)
    w1 = _wts(ab1.T, src, dst, _HEADS)
    num1 = _agg(tab1, w1, src, dst, 128)

    tab23, ab23 = _proj23(num1, b1, W2, W3, as2, ad2, as3, ad3)
    w23 = _wts(ab23.T, src, dst, 2)
    num23 = _agg(tab23, w23, src, dst, _OUT)

    out = _fin23(num23, jnp.stack([b2, b3]))
    return out[0, :_N], out[1, :_N]
